# baseline (speedup 1.0000x reference)
import functools

import jax
import jax.numpy as jnp
from jax import lax
from jax.experimental import pallas as pl
from jax.experimental.pallas import tpu as pltpu
from jax.experimental.pallas import tpu_sc as plsc

N = 10000
E = 320000
D = 128

NC = 2
NS = 16
NW = NC * NS
CH = 128
NCHT = 80
NCH_LAST = 20
NCH_FULL = 80
PH = 40
EPAD = NW * NCHT * CH
ROW0 = 624
ROWL = N - 15 * ROW0

@functools.cache
def _make_sc_aggregate():
  mesh = plsc.VectorSubcoreMesh(core_axis_name="c", subcore_axis_name="s",
                                num_cores=NC, num_subcores=NS)

  @functools.partial(
      pl.kernel,
      out_type=jax.ShapeDtypeStruct((NC, N, D), jnp.float32),
      mesh=mesh,
      scratch_types=[
          pltpu.VMEM((NCHT, CH), jnp.int32),
          pltpu.VMEM((PH, CH), jnp.int32),
          pltpu.VMEM((2, CH, D), jnp.float32),
          pltpu.VMEM_SHARED((N, D), jnp.float32),
          pltpu.SemaphoreType.DMA,
          pltpu.SemaphoreType.DMA,
      ],
  )
  def _sc_aggregate(h_hbm, src_hbm, dst_hbm, zeros_hbm, out_hbm,
                    src_v, dst_v, rows_v, aggr_sh, gsem, isem):
      cid = lax.axis_index("c")
      sid = lax.axis_index("s")
      wid = cid * NS + sid
      ncht = jnp.where(wid == NW - 1, NCH_LAST, NCH_FULL)

      base = pl.multiple_of(sid * ROW0, 8)
      rows0 = pl.ds(base, ROW0)
      rowsl = pl.ds((NS - 1) * ROW0, ROWL)

      @pl.when((cid == 0) & (sid < NS - 1))
      def _():
          pltpu.async_copy(h_hbm.at[rows0], aggr_sh.at[rows0], isem)

      @pl.when((cid == 0) & (sid == NS - 1))
      def _():
          pltpu.async_copy(h_hbm.at[rowsl], aggr_sh.at[rowsl], isem)

      @pl.when((cid == 1) & (sid < NS - 1))
      def _():
          pltpu.async_copy(zeros_hbm.at[rows0], aggr_sh.at[rows0], isem)

      @pl.when((cid == 1) & (sid == NS - 1))
      def _():
          pltpu.async_copy(zeros_hbm.at[rowsl], aggr_sh.at[rowsl], isem)

      pltpu.sync_copy(src_hbm.at[wid], src_v)

      @pl.when(sid < NS - 1)
      def _():
          pltpu.make_async_copy(zeros_hbm.at[rows0], aggr_sh.at[rows0],
                                isem).wait()

      @pl.when(sid == NS - 1)
      def _():
          pltpu.make_async_copy(zeros_hbm.at[rowsl], aggr_sh.at[rowsl],
                                isem).wait()

      plsc.subcore_barrier()

      for phase in range(2):
          start = phase * PH
          cnt = jnp.clip(ncht - start, 0, PH)

          pltpu.sync_copy(dst_hbm.at[wid, pl.ds(start, PH)], dst_v)

          @pl.when(cnt > 0)
          def _():
              pltpu.async_copy(h_hbm.at[src_v.at[start]], rows_v.at[0], gsem)

          @pl.when(cnt > 1)
          def _():
              pltpu.async_copy(h_hbm.at[src_v.at[start + 1]], rows_v.at[1],
                               gsem)

          @pl.loop(0, cnt)
          def _chunks(c):
              slot = lax.rem(c, 2)
              pltpu.make_async_copy(h_hbm.at[src_v.at[start + c]],
                                    rows_v.at[slot], gsem).wait()
              pltpu.sync_copy(rows_v.at[slot], aggr_sh.at[dst_v.at[c]], add=True)

              @pl.when(c + 2 < cnt)
              def _():
                  pltpu.async_copy(h_hbm.at[src_v.at[start + c + 2]],
                                   rows_v.at[slot], gsem)

      plsc.subcore_barrier()

      @pl.when(sid < NS - 1)
      def _():
          pltpu.sync_copy(aggr_sh.at[pl.ds(base, ROW0)],
                          out_hbm.at[cid, pl.ds(base, ROW0)])

      @pl.when(sid == NS - 1)
      def _():
          pltpu.sync_copy(aggr_sh.at[pl.ds((NS - 1) * ROW0, ROWL)],
                          out_hbm.at[cid, pl.ds((NS - 1) * ROW0, ROWL)])

  return _sc_aggregate


BR = 1000
NB = N // BR


def _bn_scale_shift(s_ref, s2_ref, g_ref, be_ref):
    mu = s_ref[...] * (1.0 / N)
    var = s2_ref[...] * (1.0 / N) - mu * mu
    scale = g_ref[...] * lax.rsqrt(var + 1e-5)
    return scale, be_ref[...] - mu * scale


def _accum_stats(t, s_ref, s2_ref, first):
    @pl.when(first)
    def _():
        s_ref[...] = jnp.zeros_like(s_ref)
        s2_ref[...] = jnp.zeros_like(s2_ref)

    s_ref[...] += jnp.sum(t, axis=0, keepdims=True)
    s2_ref[...] += jnp.sum(t * t, axis=0, keepdims=True)


def _layer_body(a_ref, w1_ref, b1_ref, g_ref, be_ref, w2_ref, b2_ref,
                o_ref, t_ref, s_ref, s2_ref):
    p = pl.program_id(0)
    i = pl.program_id(1)
    rows = pl.ds(i * BR, BR)

    @pl.when(p == 0)
    def _():
        m = a_ref[0] + a_ref[1]
        t = jnp.dot(m, w1_ref[...], preferred_element_type=jnp.float32)
        t = t + b1_ref[...]
        t_ref[rows, :] = t
        _accum_stats(t, s_ref, s2_ref, i == 0)

    @pl.when(p == 1)
    def _():
        scale, shift = _bn_scale_shift(s_ref, s2_ref, g_ref, be_ref)
        u = jnp.maximum(t_ref[rows, :] * scale + shift, 0.0)
        o = jnp.dot(u, w2_ref[...], preferred_element_type=jnp.float32)
        o_ref[...] = jnp.maximum(o + b2_ref[...], 0.0)


def _layer2_final_body(a_ref, w1_ref, b1_ref, g_ref, be_ref, w2_ref,
                       b2_ref, wf1_ref, bf1_ref, gf_ref, bef_ref, wf2_ref,
                       bf2_ref, o_ref, t_ref, h2_ref, s_ref, s2_ref):
    p = pl.program_id(0)
    i = pl.program_id(1)
    rows = pl.ds(i * BR, BR)

    @pl.when(p == 0)
    def _():
        m = a_ref[0] + a_ref[1]
        t = jnp.dot(m, w1_ref[...], preferred_element_type=jnp.float32)
        t = t + b1_ref[...]
        t_ref[rows, :] = t
        _accum_stats(t, s_ref, s2_ref, i == 0)

    @pl.when(p == 1)
    def _():
        scale, shift = _bn_scale_shift(s_ref, s2_ref, g_ref, be_ref)
        u = jnp.maximum(t_ref[rows, :] * scale + shift, 0.0)
        o = jnp.dot(u, w2_ref[...], preferred_element_type=jnp.float32)
        h2_ref[rows, :] = jnp.maximum(o + b2_ref[...], 0.0)

    @pl.when(p == 2)
    def _():
        t = jnp.dot(h2_ref[rows, :], wf1_ref[...],
                    preferred_element_type=jnp.float32)
        t = t + bf1_ref[...]
        t_ref[rows, :] = t
        _accum_stats(t, s_ref, s2_ref, i == 0)

    @pl.when(p == 3)
    def _():
        scale, shift = _bn_scale_shift(s_ref, s2_ref, gf_ref, bef_ref)
        u = jnp.maximum(t_ref[rows, :] * scale + shift, 0.0)
        o = jnp.dot(u, wf2_ref[...], preferred_element_type=jnp.float32)
        o_ref[...] = o + bf2_ref[...]


def _w_spec():
    return pl.BlockSpec((D, D), lambda p, i: (0, 0))


def _v_spec():
    return pl.BlockSpec((1, D), lambda p, i: (0, 0))


_layer_call = pl.pallas_call(
    _layer_body,
    grid=(2, NB),
    in_specs=[
        pl.BlockSpec((NC, BR, D), lambda p, i: (0, i * (1 - p), 0)),
        _w_spec(), _v_spec(), _v_spec(), _v_spec(), _w_spec(), _v_spec(),
    ],
    out_specs=pl.BlockSpec((BR, D), lambda p, i: (i * p, 0)),
    out_shape=jax.ShapeDtypeStruct((N, D), jnp.float32),
    scratch_shapes=[pltpu.VMEM((N, D), jnp.float32),
                    pltpu.VMEM((1, D), jnp.float32),
                    pltpu.VMEM((1, D), jnp.float32)],
)

_layer2_final_call = pl.pallas_call(
    _layer2_final_body,
    grid=(4, NB),
    in_specs=[
        pl.BlockSpec((NC, BR, D), lambda p, i: (0, i * (p == 0), 0)),
        _w_spec(), _v_spec(), _v_spec(), _v_spec(), _w_spec(), _v_spec(),
        _w_spec(), _v_spec(), _v_spec(), _v_spec(), _w_spec(), _v_spec(),
    ],
    out_specs=pl.BlockSpec((BR, D), lambda p, i: (i * (p == 3), 0)),
    out_shape=jax.ShapeDtypeStruct((N, D), jnp.float32),
    scratch_shapes=[pltpu.VMEM((N, D), jnp.float32),
                    pltpu.VMEM((N, D), jnp.float32),
                    pltpu.VMEM((1, D), jnp.float32),
                    pltpu.VMEM((1, D), jnp.float32)],
)


def kernel(x, edge_index, W1_0, b1_0, g_0, be_0, W2_0, b2_0,
           W1_1, b1_1, g_1, be_1, W2_1, b2_1, Wf1, bf1, gf, bef, Wf2, bf2):
    pad = jnp.zeros((EPAD - E,), jnp.int32)
    src = jnp.concatenate([edge_index[0], pad]).reshape(NW, NCHT, CH)
    dst = jnp.concatenate([edge_index[1], pad]).reshape(NW, NCHT, CH)
    zeros = jnp.zeros((N, D), jnp.float32)

    sc_aggregate = _make_sc_aggregate()
    parts = sc_aggregate(x, src, dst, zeros)
    h1 = _layer_call(parts, W1_0, b1_0.reshape(1, D), g_0.reshape(1, D),
                     be_0.reshape(1, D), W2_0, b2_0.reshape(1, D))
    parts = sc_aggregate(h1, src, dst, zeros)
    return _layer2_final_call(
        parts, W1_1, b1_1.reshape(1, D), g_1.reshape(1, D),
        be_1.reshape(1, D), W2_1, b2_1.reshape(1, D),
        Wf1, bf1.reshape(1, D), gf.reshape(1, D), bef.reshape(1, D),
        Wf2, bf2.reshape(1, D))


# speedup vs baseline: 1.0317x; 1.0317x over previous
import functools

import jax
import jax.numpy as jnp
from jax import lax
from jax.experimental import pallas as pl
from jax.experimental.pallas import tpu as pltpu
from jax.experimental.pallas import tpu_sc as plsc

N = 10000
E = 320000
D = 128

NC = 2
NS = 16
NW = NC * NS
CH = 128
NCHT = 80
NCH_LAST = 20
NCH_FULL = 80
PH = 40
NCHK = E // CH
ROW0 = 624
ROWL = N - 15 * ROW0

@functools.cache
def _make_sc_aggregate():
  mesh = plsc.VectorSubcoreMesh(core_axis_name="c", subcore_axis_name="s",
                                num_cores=NC, num_subcores=NS)

  @functools.partial(
      pl.kernel,
      out_type=jax.ShapeDtypeStruct((NC, N, D), jnp.float32),
      mesh=mesh,
      scratch_types=[
          pltpu.VMEM((NCHT, CH), jnp.int32),
          pltpu.VMEM((PH, CH), jnp.int32),
          pltpu.VMEM((2, CH, D), jnp.float32),
          pltpu.VMEM_SHARED((N, D), jnp.float32),
          pltpu.SemaphoreType.DMA,
          pltpu.SemaphoreType.DMA,
      ],
  )
  def _sc_aggregate(h_hbm, src_hbm, dst_hbm, zeros_hbm, out_hbm,
                    src_v, dst_v, rows_v, aggr_sh, gsem, isem):
      cid = lax.axis_index("c")
      sid = lax.axis_index("s")
      wid = cid * NS + sid
      ncht = jnp.where(wid == NW - 1, NCH_LAST, NCH_FULL)

      base = pl.multiple_of(sid * ROW0, 8)
      rows0 = pl.ds(base, ROW0)
      rowsl = pl.ds((NS - 1) * ROW0, ROWL)

      @pl.when((cid == 0) & (sid < NS - 1))
      def _():
          pltpu.async_copy(h_hbm.at[rows0], aggr_sh.at[rows0], isem)

      @pl.when((cid == 0) & (sid == NS - 1))
      def _():
          pltpu.async_copy(h_hbm.at[rowsl], aggr_sh.at[rowsl], isem)

      @pl.when((cid == 1) & (sid < NS - 1))
      def _():
          pltpu.async_copy(zeros_hbm.at[rows0], aggr_sh.at[rows0], isem)

      @pl.when((cid == 1) & (sid == NS - 1))
      def _():
          pltpu.async_copy(zeros_hbm.at[rowsl], aggr_sh.at[rowsl], isem)

      cbase = pl.multiple_of(wid * NCHT, 8)

      @pl.when(wid < NW - 1)
      def _():
          pltpu.sync_copy(src_hbm.at[pl.ds(cbase, NCHT)], src_v)

      @pl.when(wid == NW - 1)
      def _():
          pltpu.sync_copy(src_hbm.at[pl.ds((NW - 1) * NCHT, NCH_LAST)],
                          src_v.at[pl.ds(0, NCH_LAST)])

      @pl.when(sid < NS - 1)
      def _():
          pltpu.make_async_copy(zeros_hbm.at[rows0], aggr_sh.at[rows0],
                                isem).wait()

      @pl.when(sid == NS - 1)
      def _():
          pltpu.make_async_copy(zeros_hbm.at[rowsl], aggr_sh.at[rowsl],
                                isem).wait()

      plsc.subcore_barrier()

      for phase in range(2):
          start = phase * PH
          cnt = jnp.clip(ncht - start, 0, PH)

          @pl.when(wid < NW - 1)
          def _():
              pltpu.sync_copy(dst_hbm.at[pl.ds(cbase + start, PH)], dst_v)

          if phase == 0:
              @pl.when(wid == NW - 1)
              def _():
                  pltpu.sync_copy(
                      dst_hbm.at[pl.ds((NW - 1) * NCHT, NCH_LAST)],
                      dst_v.at[pl.ds(0, NCH_LAST)])

          @pl.when(cnt > 0)
          def _():
              pltpu.async_copy(h_hbm.at[src_v.at[start]], rows_v.at[0], gsem)

          @pl.when(cnt > 1)
          def _():
              pltpu.async_copy(h_hbm.at[src_v.at[start + 1]], rows_v.at[1],
                               gsem)

          @pl.loop(0, cnt)
          def _chunks(c):
              slot = lax.rem(c, 2)
              pltpu.make_async_copy(h_hbm.at[src_v.at[start + c]],
                                    rows_v.at[slot], gsem).wait()
              pltpu.sync_copy(rows_v.at[slot], aggr_sh.at[dst_v.at[c]], add=True)

              @pl.when(c + 2 < cnt)
              def _():
                  pltpu.async_copy(h_hbm.at[src_v.at[start + c + 2]],
                                   rows_v.at[slot], gsem)

      plsc.subcore_barrier()

      @pl.when(sid < NS - 1)
      def _():
          pltpu.sync_copy(aggr_sh.at[pl.ds(base, ROW0)],
                          out_hbm.at[cid, pl.ds(base, ROW0)])

      @pl.when(sid == NS - 1)
      def _():
          pltpu.sync_copy(aggr_sh.at[pl.ds((NS - 1) * ROW0, ROWL)],
                          out_hbm.at[cid, pl.ds((NS - 1) * ROW0, ROWL)])

  return _sc_aggregate


BR = 1000
NB = N // BR


def _mlp(read_m, w_refs, t_ref, write_o, relu_out):
    w1_ref, b1_ref, g_ref, be_ref, w2_ref, b2_ref = w_refs
    w1 = w1_ref[...]
    b1 = b1_ref[...]
    w2 = w2_ref[...]
    b2 = b2_ref[...]

    def pass1(i, carry):
        s, s2 = carry
        rows = pl.ds(i * BR, BR)
        t = jnp.dot(read_m(rows), w1, preferred_element_type=jnp.float32) + b1
        t_ref[rows, :] = t
        return (s + jnp.sum(t, axis=0, keepdims=True),
                s2 + jnp.sum(t * t, axis=0, keepdims=True))

    zero = jnp.zeros((1, D), jnp.float32)
    s, s2 = lax.fori_loop(0, NB, pass1, (zero, zero))
    mu = s * (1.0 / N)
    var = s2 * (1.0 / N) - mu * mu
    rstd = lax.rsqrt(var + 1e-5)
    scale = g_ref[...] * rstd
    shift = be_ref[...] - mu * scale

    def pass2(i, _):
        rows = pl.ds(i * BR, BR)
        u = jnp.maximum(t_ref[rows, :] * scale + shift, 0.0)
        o = jnp.dot(u, w2, preferred_element_type=jnp.float32) + b2
        if relu_out:
            o = jnp.maximum(o, 0.0)
        write_o(rows, o)
        return 0

    lax.fori_loop(0, NB, pass2, 0)


def _layer_body(a_ref, w1_ref, b1_ref, g_ref, be_ref, w2_ref, b2_ref,
                o_ref, t_ref):
    def read_m(rows):
        return a_ref[0, rows, :] + a_ref[1, rows, :]

    def write_o(rows, o):
        o_ref[rows, :] = o

    _mlp(read_m, (w1_ref, b1_ref, g_ref, be_ref, w2_ref, b2_ref),
         t_ref, write_o, relu_out=True)


def _layer2_final_body(a_ref, w1_ref, b1_ref, g_ref, be_ref, w2_ref,
                       b2_ref, wf1_ref, bf1_ref, gf_ref, bef_ref, wf2_ref,
                       bf2_ref, o_ref, t_ref, h2_ref):
    def read_m(rows):
        return a_ref[0, rows, :] + a_ref[1, rows, :]

    def write_h2(rows, o):
        h2_ref[rows, :] = o

    _mlp(read_m, (w1_ref, b1_ref, g_ref, be_ref, w2_ref, b2_ref),
         t_ref, write_h2, relu_out=True)

    def read_h2(rows):
        return h2_ref[rows, :]

    def write_o(rows, o):
        o_ref[rows, :] = o

    _mlp(read_h2, (wf1_ref, bf1_ref, gf_ref, bef_ref, wf2_ref, bf2_ref),
         t_ref, write_o, relu_out=False)


_layer_call = pl.pallas_call(
    _layer_body,
    out_shape=jax.ShapeDtypeStruct((N, D), jnp.float32),
    scratch_shapes=[pltpu.VMEM((N, D), jnp.float32)],
)

_layer2_final_call = pl.pallas_call(
    _layer2_final_body,
    out_shape=jax.ShapeDtypeStruct((N, D), jnp.float32),
    scratch_shapes=[pltpu.VMEM((N, D), jnp.float32),
                    pltpu.VMEM((N, D), jnp.float32)],
)


def kernel(x, edge_index, W1_0, b1_0, g_0, be_0, W2_0, b2_0,
           W1_1, b1_1, g_1, be_1, W2_1, b2_1, Wf1, bf1, gf, bef, Wf2, bf2):
    src = edge_index[0].reshape(NCHK, CH)
    dst = edge_index[1].reshape(NCHK, CH)
    zeros = jnp.zeros((N, D), jnp.float32)

    sc_aggregate = _make_sc_aggregate()
    parts = sc_aggregate(x, src, dst, zeros)
    h1 = _layer_call(parts, W1_0, b1_0.reshape(1, D), g_0.reshape(1, D),
                     be_0.reshape(1, D), W2_0, b2_0.reshape(1, D))
    parts = sc_aggregate(h1, src, dst, zeros)
    return _layer2_final_call(
        parts, W1_1, b1_1.reshape(1, D), g_1.reshape(1, D),
        be_1.reshape(1, D), W2_1, b2_1.reshape(1, D),
        Wf1, bf1.reshape(1, D), gf.reshape(1, D), bef.reshape(1, D),
        Wf2, bf2.reshape(1, D))


# speedup vs baseline: 1.0649x; 1.0321x over previous
import functools

import jax
import jax.numpy as jnp
from jax import lax
from jax.experimental import pallas as pl
from jax.experimental.pallas import tpu as pltpu
from jax.experimental.pallas import tpu_sc as plsc

N = 10000
E = 320000
D = 128

NC = 2
NS = 16
NW = NC * NS
CH = 128
NCHT = 80
NCH_LAST = 20
NCH_FULL = 80
PH = 40
NCHK = E // CH
ROW0 = 624
ROWL = N - 15 * ROW0

@functools.cache
def _make_sc_aggregate():
  mesh = plsc.VectorSubcoreMesh(core_axis_name="c", subcore_axis_name="s",
                                num_cores=NC, num_subcores=NS)

  @functools.partial(
      pl.kernel,
      out_type=jax.ShapeDtypeStruct((NC, N, D), jnp.float32),
      mesh=mesh,
      scratch_types=[
          pltpu.VMEM((NCHT, CH), jnp.int32),
          pltpu.VMEM((PH, CH), jnp.int32),
          pltpu.VMEM((2, CH, D), jnp.float32),
          pltpu.VMEM_SHARED((N, D), jnp.float32),
          pltpu.SemaphoreType.DMA,
          pltpu.SemaphoreType.DMA,
      ],
  )
  def _sc_aggregate(h_hbm, src_hbm, dst_hbm, zeros_hbm, out_hbm,
                    src_v, dst_v, rows_v, aggr_sh, gsem, isem):
      cid = lax.axis_index("c")
      sid = lax.axis_index("s")
      wid = cid * NS + sid
      ncht = jnp.where(wid == NW - 1, NCH_LAST, NCH_FULL)

      base = pl.multiple_of(sid * ROW0, 8)
      rows0 = pl.ds(base, ROW0)
      rowsl = pl.ds((NS - 1) * ROW0, ROWL)

      @pl.when((cid == 0) & (sid < NS - 1))
      def _():
          pltpu.async_copy(h_hbm.at[rows0], aggr_sh.at[rows0], isem)

      @pl.when((cid == 0) & (sid == NS - 1))
      def _():
          pltpu.async_copy(h_hbm.at[rowsl], aggr_sh.at[rowsl], isem)

      @pl.when((cid == 1) & (sid < NS - 1))
      def _():
          pltpu.async_copy(zeros_hbm.at[rows0], aggr_sh.at[rows0], isem)

      @pl.when((cid == 1) & (sid == NS - 1))
      def _():
          pltpu.async_copy(zeros_hbm.at[rowsl], aggr_sh.at[rowsl], isem)

      cbase = pl.multiple_of(wid * NCHT, 8)

      @pl.when(wid < NW - 1)
      def _():
          pltpu.sync_copy(src_hbm.at[pl.ds(cbase, NCHT)], src_v)

      @pl.when(wid == NW - 1)
      def _():
          pltpu.sync_copy(src_hbm.at[pl.ds((NW - 1) * NCHT, NCH_LAST)],
                          src_v.at[pl.ds(0, NCH_LAST)])

      @pl.when(sid < NS - 1)
      def _():
          pltpu.make_async_copy(zeros_hbm.at[rows0], aggr_sh.at[rows0],
                                isem).wait()

      @pl.when(sid == NS - 1)
      def _():
          pltpu.make_async_copy(zeros_hbm.at[rowsl], aggr_sh.at[rowsl],
                                isem).wait()

      plsc.subcore_barrier()

      for phase in range(2):
          start = phase * PH
          cnt = jnp.clip(ncht - start, 0, PH)

          @pl.when(wid < NW - 1)
          def _():
              pltpu.sync_copy(dst_hbm.at[pl.ds(cbase + start, PH)], dst_v)

          if phase == 0:
              @pl.when(wid == NW - 1)
              def _():
                  pltpu.sync_copy(
                      dst_hbm.at[pl.ds((NW - 1) * NCHT, NCH_LAST)],
                      dst_v.at[pl.ds(0, NCH_LAST)])

          @pl.when(cnt > 0)
          def _():
              pltpu.async_copy(h_hbm.at[src_v.at[start]], rows_v.at[0], gsem)

          @pl.when(cnt > 1)
          def _():
              pltpu.async_copy(h_hbm.at[src_v.at[start + 1]], rows_v.at[1],
                               gsem)

          @pl.loop(0, cnt)
          def _chunks(c):
              slot = lax.rem(c, 2)
              pltpu.make_async_copy(h_hbm.at[src_v.at[start + c]],
                                    rows_v.at[slot], gsem).wait()
              pltpu.sync_copy(rows_v.at[slot], aggr_sh.at[dst_v.at[c]], add=True)

              @pl.when(c + 2 < cnt)
              def _():
                  pltpu.async_copy(h_hbm.at[src_v.at[start + c + 2]],
                                   rows_v.at[slot], gsem)

      plsc.subcore_barrier()

      @pl.when(sid < NS - 1)
      def _():
          pltpu.sync_copy(aggr_sh.at[pl.ds(base, ROW0)],
                          out_hbm.at[cid, pl.ds(base, ROW0)])

      @pl.when(sid == NS - 1)
      def _():
          pltpu.sync_copy(aggr_sh.at[pl.ds((NS - 1) * ROW0, ROWL)],
                          out_hbm.at[cid, pl.ds((NS - 1) * ROW0, ROWL)])

  return _sc_aggregate


BR = 2000
NB = N // BR


def _mlp(read_m, w_refs, t_ref, write_o, relu_out):
    w1_ref, b1_ref, g_ref, be_ref, w2_ref, b2_ref = w_refs
    w1 = w1_ref[...]
    b1 = b1_ref[...]
    w2 = w2_ref[...]
    b2 = b2_ref[...]

    def pass1(i, carry):
        s, s2 = carry
        rows = pl.ds(i * BR, BR)
        t = jnp.dot(read_m(rows), w1, preferred_element_type=jnp.float32) + b1
        t_ref[rows, :] = t
        return (s + jnp.sum(t, axis=0, keepdims=True),
                s2 + jnp.sum(t * t, axis=0, keepdims=True))

    zero = jnp.zeros((1, D), jnp.float32)
    s, s2 = lax.fori_loop(0, NB, pass1, (zero, zero))
    mu = s * (1.0 / N)
    var = s2 * (1.0 / N) - mu * mu
    rstd = lax.rsqrt(var + 1e-5)
    scale = g_ref[...] * rstd
    shift = be_ref[...] - mu * scale

    def pass2(i, _):
        rows = pl.ds(i * BR, BR)
        u = jnp.maximum(t_ref[rows, :] * scale + shift, 0.0)
        o = jnp.dot(u, w2, preferred_element_type=jnp.float32) + b2
        if relu_out:
            o = jnp.maximum(o, 0.0)
        write_o(rows, o)
        return 0

    lax.fori_loop(0, NB, pass2, 0)


def _layer_body(a_ref, w1_ref, b1_ref, g_ref, be_ref, w2_ref, b2_ref,
                o_ref, t_ref):
    def read_m(rows):
        return a_ref[0, rows, :] + a_ref[1, rows, :]

    def write_o(rows, o):
        o_ref[rows, :] = o

    _mlp(read_m, (w1_ref, b1_ref, g_ref, be_ref, w2_ref, b2_ref),
         t_ref, write_o, relu_out=True)


def _layer2_final_body(a_ref, w1_ref, b1_ref, g_ref, be_ref, w2_ref,
                       b2_ref, wf1_ref, bf1_ref, gf_ref, bef_ref, wf2_ref,
                       bf2_ref, o_ref, t_ref, h2_ref):
    def read_m(rows):
        return a_ref[0, rows, :] + a_ref[1, rows, :]

    def write_h2(rows, o):
        h2_ref[rows, :] = o

    _mlp(read_m, (w1_ref, b1_ref, g_ref, be_ref, w2_ref, b2_ref),
         t_ref, write_h2, relu_out=True)

    def read_h2(rows):
        return h2_ref[rows, :]

    def write_o(rows, o):
        o_ref[rows, :] = o

    _mlp(read_h2, (wf1_ref, bf1_ref, gf_ref, bef_ref, wf2_ref, bf2_ref),
         t_ref, write_o, relu_out=False)


_layer_call = pl.pallas_call(
    _layer_body,
    out_shape=jax.ShapeDtypeStruct((N, D), jnp.float32),
    scratch_shapes=[pltpu.VMEM((N, D), jnp.float32)],
)

_layer2_final_call = pl.pallas_call(
    _layer2_final_body,
    out_shape=jax.ShapeDtypeStruct((N, D), jnp.float32),
    scratch_shapes=[pltpu.VMEM((N, D), jnp.float32),
                    pltpu.VMEM((N, D), jnp.float32)],
)


def kernel(x, edge_index, W1_0, b1_0, g_0, be_0, W2_0, b2_0,
           W1_1, b1_1, g_1, be_1, W2_1, b2_1, Wf1, bf1, gf, bef, Wf2, bf2):
    src = edge_index[0].reshape(NCHK, CH)
    dst = edge_index[1].reshape(NCHK, CH)
    zeros = jnp.zeros((N, D), jnp.float32)

    sc_aggregate = _make_sc_aggregate()
    parts = sc_aggregate(x, src, dst, zeros)
    h1 = _layer_call(parts, W1_0, b1_0.reshape(1, D), g_0.reshape(1, D),
                     be_0.reshape(1, D), W2_0, b2_0.reshape(1, D))
    parts = sc_aggregate(h1, src, dst, zeros)
    return _layer2_final_call(
        parts, W1_1, b1_1.reshape(1, D), g_1.reshape(1, D),
        be_1.reshape(1, D), W2_1, b2_1.reshape(1, D),
        Wf1, bf1.reshape(1, D), gf.reshape(1, D), bef.reshape(1, D),
        Wf2, bf2.reshape(1, D))


# speedup vs baseline: 1.0650x; 1.0001x over previous
import functools

import jax
import jax.numpy as jnp
from jax import lax
from jax.experimental import pallas as pl
from jax.experimental.pallas import tpu as pltpu
from jax.experimental.pallas import tpu_sc as plsc

N = 10000
E = 320000
D = 128

NC = 2
NS = 16
NW = NC * NS
CH = 128
NCHT = 80
NCH_LAST = 20
NCH_FULL = 80
PH = 40
NCHK = E // CH
ROW0 = 624
ROWL = N - 15 * ROW0

@functools.cache
def _make_sc_aggregate():
  mesh = plsc.VectorSubcoreMesh(core_axis_name="c", subcore_axis_name="s",
                                num_cores=NC, num_subcores=NS)

  @functools.partial(
      pl.kernel,
      out_type=jax.ShapeDtypeStruct((NC, N, D), jnp.float32),
      mesh=mesh,
      scratch_types=[
          pltpu.VMEM((NCHT, CH), jnp.int32),
          pltpu.VMEM((PH, CH), jnp.int32),
          pltpu.VMEM((2, CH, D), jnp.float32),
          pltpu.VMEM_SHARED((N, D), jnp.float32),
          pltpu.SemaphoreType.DMA,
          pltpu.SemaphoreType.DMA,
      ],
  )
  def _sc_aggregate(h_hbm, src_hbm, dst_hbm, zeros_hbm, out_hbm,
                    src_v, dst_v, rows_v, aggr_sh, gsem, isem):
      cid = lax.axis_index("c")
      sid = lax.axis_index("s")
      wid = cid * NS + sid
      ncht = jnp.where(wid == NW - 1, NCH_LAST, NCH_FULL)

      base = pl.multiple_of(sid * ROW0, 8)
      rows0 = pl.ds(base, ROW0)
      rowsl = pl.ds((NS - 1) * ROW0, ROWL)

      @pl.when((cid == 0) & (sid < NS - 1))
      def _():
          pltpu.async_copy(h_hbm.at[rows0], aggr_sh.at[rows0], isem)

      @pl.when((cid == 0) & (sid == NS - 1))
      def _():
          pltpu.async_copy(h_hbm.at[rowsl], aggr_sh.at[rowsl], isem)

      @pl.when((cid == 1) & (sid < NS - 1))
      def _():
          pltpu.async_copy(zeros_hbm.at[rows0], aggr_sh.at[rows0], isem)

      @pl.when((cid == 1) & (sid == NS - 1))
      def _():
          pltpu.async_copy(zeros_hbm.at[rowsl], aggr_sh.at[rowsl], isem)

      cbase = pl.multiple_of(wid * NCHT, 8)

      @pl.when(wid < NW - 1)
      def _():
          pltpu.sync_copy(src_hbm.at[pl.ds(cbase, NCHT)], src_v)

      @pl.when(wid == NW - 1)
      def _():
          pltpu.sync_copy(src_hbm.at[pl.ds((NW - 1) * NCHT, NCH_LAST)],
                          src_v.at[pl.ds(0, NCH_LAST)])

      @pl.when(sid < NS - 1)
      def _():
          pltpu.make_async_copy(zeros_hbm.at[rows0], aggr_sh.at[rows0],
                                isem).wait()

      @pl.when(sid == NS - 1)
      def _():
          pltpu.make_async_copy(zeros_hbm.at[rowsl], aggr_sh.at[rowsl],
                                isem).wait()

      plsc.subcore_barrier()

      for phase in range(2):
          start = phase * PH
          cnt = jnp.clip(ncht - start, 0, PH)

          @pl.when(wid < NW - 1)
          def _():
              pltpu.sync_copy(dst_hbm.at[pl.ds(cbase + start, PH)], dst_v)

          if phase == 0:
              @pl.when(wid == NW - 1)
              def _():
                  pltpu.sync_copy(
                      dst_hbm.at[pl.ds((NW - 1) * NCHT, NCH_LAST)],
                      dst_v.at[pl.ds(0, NCH_LAST)])

          @pl.when(cnt > 0)
          def _():
              pltpu.async_copy(h_hbm.at[src_v.at[start]], rows_v.at[0], gsem)

          @pl.when(cnt > 1)
          def _():
              pltpu.async_copy(h_hbm.at[src_v.at[start + 1]], rows_v.at[1],
                               gsem)

          @pl.loop(0, cnt)
          def _chunks(c):
              slot = lax.rem(c, 2)
              pltpu.make_async_copy(h_hbm.at[src_v.at[start + c]],
                                    rows_v.at[slot], gsem).wait()
              pltpu.sync_copy(rows_v.at[slot], aggr_sh.at[dst_v.at[c]], add=True)

              @pl.when(c + 2 < cnt)
              def _():
                  pltpu.async_copy(h_hbm.at[src_v.at[start + c + 2]],
                                   rows_v.at[slot], gsem)

      plsc.subcore_barrier()

      @pl.when(sid < NS - 1)
      def _():
          pltpu.sync_copy(aggr_sh.at[pl.ds(base, ROW0)],
                          out_hbm.at[cid, pl.ds(base, ROW0)])

      @pl.when(sid == NS - 1)
      def _():
          pltpu.sync_copy(aggr_sh.at[pl.ds((NS - 1) * ROW0, ROWL)],
                          out_hbm.at[cid, pl.ds((NS - 1) * ROW0, ROWL)])

  return _sc_aggregate


BR = 5000
NB = N // BR


def _mlp(read_m, w_refs, t_ref, write_o, relu_out):
    w1_ref, b1_ref, g_ref, be_ref, w2_ref, b2_ref = w_refs
    w1 = w1_ref[...]
    b1 = b1_ref[...]
    w2 = w2_ref[...]
    b2 = b2_ref[...]

    def pass1(i, carry):
        s, s2 = carry
        rows = pl.ds(i * BR, BR)
        t = jnp.dot(read_m(rows), w1, preferred_element_type=jnp.float32) + b1
        t_ref[rows, :] = t
        return (s + jnp.sum(t, axis=0, keepdims=True),
                s2 + jnp.sum(t * t, axis=0, keepdims=True))

    zero = jnp.zeros((1, D), jnp.float32)
    s, s2 = lax.fori_loop(0, NB, pass1, (zero, zero))
    mu = s * (1.0 / N)
    var = s2 * (1.0 / N) - mu * mu
    rstd = lax.rsqrt(var + 1e-5)
    scale = g_ref[...] * rstd
    shift = be_ref[...] - mu * scale

    def pass2(i, _):
        rows = pl.ds(i * BR, BR)
        u = jnp.maximum(t_ref[rows, :] * scale + shift, 0.0)
        o = jnp.dot(u, w2, preferred_element_type=jnp.float32) + b2
        if relu_out:
            o = jnp.maximum(o, 0.0)
        write_o(rows, o)
        return 0

    lax.fori_loop(0, NB, pass2, 0)


def _layer_body(a_ref, w1_ref, b1_ref, g_ref, be_ref, w2_ref, b2_ref,
                o_ref, t_ref):
    def read_m(rows):
        return a_ref[0, rows, :] + a_ref[1, rows, :]

    def write_o(rows, o):
        o_ref[rows, :] = o

    _mlp(read_m, (w1_ref, b1_ref, g_ref, be_ref, w2_ref, b2_ref),
         t_ref, write_o, relu_out=True)


def _layer2_final_body(a_ref, w1_ref, b1_ref, g_ref, be_ref, w2_ref,
                       b2_ref, wf1_ref, bf1_ref, gf_ref, bef_ref, wf2_ref,
                       bf2_ref, o_ref, t_ref, h2_ref):
    def read_m(rows):
        return a_ref[0, rows, :] + a_ref[1, rows, :]

    def write_h2(rows, o):
        h2_ref[rows, :] = o

    _mlp(read_m, (w1_ref, b1_ref, g_ref, be_ref, w2_ref, b2_ref),
         t_ref, write_h2, relu_out=True)

    def read_h2(rows):
        return h2_ref[rows, :]

    def write_o(rows, o):
        o_ref[rows, :] = o

    _mlp(read_h2, (wf1_ref, bf1_ref, gf_ref, bef_ref, wf2_ref, bf2_ref),
         t_ref, write_o, relu_out=False)


_layer_call = pl.pallas_call(
    _layer_body,
    out_shape=jax.ShapeDtypeStruct((N, D), jnp.float32),
    scratch_shapes=[pltpu.VMEM((N, D), jnp.float32)],
)

_layer2_final_call = pl.pallas_call(
    _layer2_final_body,
    out_shape=jax.ShapeDtypeStruct((N, D), jnp.float32),
    scratch_shapes=[pltpu.VMEM((N, D), jnp.float32),
                    pltpu.VMEM((N, D), jnp.float32)],
)


def kernel(x, edge_index, W1_0, b1_0, g_0, be_0, W2_0, b2_0,
           W1_1, b1_1, g_1, be_1, W2_1, b2_1, Wf1, bf1, gf, bef, Wf2, bf2):
    src = edge_index[0].reshape(NCHK, CH)
    dst = edge_index[1].reshape(NCHK, CH)
    zeros = jnp.zeros((N, D), jnp.float32)

    sc_aggregate = _make_sc_aggregate()
    parts = sc_aggregate(x, src, dst, zeros)
    h1 = _layer_call(parts, W1_0, b1_0.reshape(1, D), g_0.reshape(1, D),
                     be_0.reshape(1, D), W2_0, b2_0.reshape(1, D))
    parts = sc_aggregate(h1, src, dst, zeros)
    return _layer2_final_call(
        parts, W1_1, b1_1.reshape(1, D), g_1.reshape(1, D),
        be_1.reshape(1, D), W2_1, b2_1.reshape(1, D),
        Wf1, bf1.reshape(1, D), gf.reshape(1, D), bef.reshape(1, D),
        Wf2, bf2.reshape(1, D))
